# baseline (speedup 1.0000x reference)
import functools

import jax
import jax.numpy as jnp
from jax import lax
from jax.experimental import pallas as pl
from jax.experimental.pallas import tpu as pltpu
from jax.experimental.pallas import tpu_sc as plsc

_N = 10000
_E = 320000
_D = 128
_G = 64
_BN_EPS = 1e-5

_NC = 2
_NS = 16
_NW = _NC * _NS
_CHUNK = 125
_EPW = _E // _NW
_NCHUNK = _EPW // _CHUNK
_NP = 10240
_RPS = _NP // _NS


@functools.lru_cache(maxsize=None)
def _make_seg_sum(F):
  mesh = plsc.VectorSubcoreMesh(core_axis_name="c", subcore_axis_name="s",
                                num_cores=_NC, num_subcores=_NS)

  @functools.partial(
      pl.kernel,
      out_type=jax.ShapeDtypeStruct((_NC, _NP, F), jnp.float32),
      mesh=mesh,
      compiler_params=pltpu.CompilerParams(use_tc_tiling_on_sc=False),
      scratch_types=[
          pltpu.VMEM((_NCHUNK, _CHUNK), jnp.int32),
          pltpu.VMEM((_NCHUNK, _CHUNK), jnp.int32),
          pltpu.VMEM((2, _CHUNK, F), jnp.float32),
          pltpu.VMEM_SHARED((_NP, F), jnp.float32),
          pltpu.SemaphoreType.DMA,
          pltpu.SemaphoreType.DMA,
      ],
  )
  def seg_sum(feat, src2d, dst2d, zeros, out, src_v, dst_v, rows_v, acc,
              sem0, sem1):
    cid = lax.axis_index("c")
    sid = lax.axis_index("s")
    wid = sid * _NC + cid
    pltpu.sync_copy(zeros.at[pl.ds(sid * _RPS, _RPS)],
                    acc.at[pl.ds(sid * _RPS, _RPS)])
    pltpu.sync_copy(src2d.at[pl.ds(wid * _NCHUNK, _NCHUNK)], src_v)
    pltpu.sync_copy(dst2d.at[pl.ds(wid * _NCHUNK, _NCHUNK)], dst_v)
    plsc.subcore_barrier()

    sems = (sem0, sem1)
    for b in range(2):
      pltpu.async_copy(feat.at[src_v.at[b]], rows_v.at[b], sems[b])

    def body(j, carry):
      for b in range(2):

        @pl.when(lax.rem(j, 2) == b)
        def _():
          pltpu.make_async_copy(feat.at[src_v.at[j]], rows_v.at[b],
                                sems[b]).wait()
          pltpu.sync_copy(rows_v.at[b], acc.at[dst_v.at[j]], add=True)

          @pl.when(j + 2 < _NCHUNK)
          def _():
            pltpu.async_copy(feat.at[src_v.at[j + 2]], rows_v.at[b], sems[b])

      return carry

    lax.fori_loop(0, _NCHUNK, body, 0)
    plsc.subcore_barrier()
    pltpu.sync_copy(acc.at[pl.ds(sid * _RPS, _RPS)],
                    out.at[cid, pl.ds(sid * _RPS, _RPS)])

  return seg_sum


def _proj_body(x_ref, w_ref, o_ref):
  o_ref[...] = jnp.dot(x_ref[...], w_ref[...],
                       preferred_element_type=jnp.float32, precision=jax.lax.Precision.HIGHEST)


def _proj(x, W1):
  return pl.pallas_call(
      _proj_body,
      grid=(10,),
      in_specs=[
          pl.BlockSpec((_N // 10, _D), lambda i: (i, 0)),
          pl.BlockSpec((_D, 32), lambda i: (0, 0)),
      ],
      out_specs=pl.BlockSpec((_N // 10, 32), lambda i: (i, 0)),
      out_shape=jax.ShapeDtypeStruct((_N, 32), jnp.float32),
  )(x, W1)


def _mlp1_body(xp_ref, parts_ref, b1_ref, w2_ref, b2_ref, g1_ref, be1_ref,
               rm1_ref, rv1_ref, o_ref):
  t = xp_ref[...] + parts_ref[0] + parts_ref[1] + b1_ref[...]
  t = jnp.maximum(t, 0.0)
  t = jnp.dot(t, w2_ref[...], preferred_element_type=jnp.float32, precision=jax.lax.Precision.HIGHEST) + b2_ref[...]
  t = jnp.maximum(t, 0.0)
  scale = g1_ref[...] * lax.rsqrt(rv1_ref[...] + _BN_EPS)
  t = (t - rm1_ref[...]) * scale + be1_ref[...]
  o_ref[...] = jnp.maximum(t, 0.0)


def _mlp1(xp, parts, b1, W2, b2, g1, be1, rm1, rv1):
  blk = _N // 10
  return pl.pallas_call(
      _mlp1_body,
      grid=(10,),
      in_specs=[
          pl.BlockSpec((blk, 32), lambda i: (i, 0)),
          pl.BlockSpec((_NC, blk, 32), lambda i: (0, i, 0)),
          pl.BlockSpec((1, 32), lambda i: (0, 0)),
          pl.BlockSpec((32, 16), lambda i: (0, 0)),
          pl.BlockSpec((1, 16), lambda i: (0, 0)),
          pl.BlockSpec((1, 16), lambda i: (0, 0)),
          pl.BlockSpec((1, 16), lambda i: (0, 0)),
          pl.BlockSpec((1, 16), lambda i: (0, 0)),
          pl.BlockSpec((1, 16), lambda i: (0, 0)),
      ],
      out_specs=pl.BlockSpec((blk, 16), lambda i: (i, 0)),
      out_shape=jax.ShapeDtypeStruct((_N, 16), jnp.float32),
  )(xp, parts, b1, W2, b2, g1, be1, rm1, rv1)


def _mlp2_body(h_ref, parts_ref, w3_ref, b3_ref, w4_ref, b4_ref, g2_ref,
               be2_ref, rm2_ref, rv2_ref, o_ref):
  t = h_ref[...] + parts_ref[0] + parts_ref[1]
  t = jnp.dot(t, w3_ref[...], preferred_element_type=jnp.float32, precision=jax.lax.Precision.HIGHEST) + b3_ref[...]
  t = jnp.maximum(t, 0.0)
  t = jnp.dot(t, w4_ref[...], preferred_element_type=jnp.float32, precision=jax.lax.Precision.HIGHEST) + b4_ref[...]
  t = jnp.maximum(t, 0.0)
  scale = g2_ref[...] * lax.rsqrt(rv2_ref[...] + _BN_EPS)
  t = (t - rm2_ref[...]) * scale + be2_ref[...]
  o_ref[...] = jnp.maximum(t, 0.0)


def _mlp2(h, parts, W3, b3, W4, b4, g2, be2, rm2, rv2):
  blk = _N // 10
  return pl.pallas_call(
      _mlp2_body,
      grid=(10,),
      in_specs=[
          pl.BlockSpec((blk, 16), lambda i: (i, 0)),
          pl.BlockSpec((_NC, blk, 16), lambda i: (0, i, 0)),
          pl.BlockSpec((16, 16), lambda i: (0, 0)),
          pl.BlockSpec((1, 16), lambda i: (0, 0)),
          pl.BlockSpec((16, 16), lambda i: (0, 0)),
          pl.BlockSpec((1, 16), lambda i: (0, 0)),
          pl.BlockSpec((1, 16), lambda i: (0, 0)),
          pl.BlockSpec((1, 16), lambda i: (0, 0)),
          pl.BlockSpec((1, 16), lambda i: (0, 0)),
          pl.BlockSpec((1, 16), lambda i: (0, 0)),
      ],
      out_specs=pl.BlockSpec((blk, 16), lambda i: (i, 0)),
      out_shape=jax.ShapeDtypeStruct((_N, 16), jnp.float32),
  )(h, parts, W3, b3, W4, b4, g2, be2, rm2, rv2)


def _pool_body(h2_ref, h2p_ref, bt_ref, bp_ref, wfa_ref, wfb_ref, bf_ref,
               o_ref):
  hp = jax.lax.Precision.HIGHEST
  h2 = h2_ref[...]
  h2p = h2p_ref[...]
  bt = bt_ref[...]
  bp = bp_ref[...]
  rows = lax.broadcasted_iota(jnp.int32, (_G, 1), 0)

  oht = (lax.broadcasted_iota(jnp.int32, (_G, _N), 0).astype(jnp.float32)
         == bt).astype(jnp.float32)
  xsum = jnp.dot(oht, h2, preferred_element_type=jnp.float32, precision=hp)
  cnt = jnp.dot(oht, jnp.ones((_N, 1), jnp.float32),
                preferred_element_type=jnp.float32, precision=hp)
  xmean = xsum / jnp.maximum(cnt, 1.0)

  def body(g, xmax):
    t = jnp.where(bp == g.astype(jnp.float32), h2p, -jnp.inf)
    r = jnp.max(t, axis=0, keepdims=True)
    r = jnp.maximum(r[:, :64], r[:, 64:])
    r = jnp.maximum(r[:, :32], r[:, 32:])
    r = jnp.maximum(r[:, :16], r[:, 16:])
    return jnp.where(rows == g, r, xmax)

  xmax = lax.fori_loop(0, _G, body,
                       jnp.full((_G, 16), -jnp.inf, jnp.float32))
  o_ref[...] = (jnp.dot(xmax, wfa_ref[...],
                        preferred_element_type=jnp.float32, precision=hp)
                + jnp.dot(xmean, wfb_ref[...],
                          preferred_element_type=jnp.float32, precision=hp)
                + bf_ref[...])


def _pool(h2, h2p, batcht, batchp, Wfa, Wfb, bf):
  return pl.pallas_call(
      _pool_body,
      out_shape=jax.ShapeDtypeStruct((_G, 2), jnp.float32),
  )(h2, h2p, batcht, batchp, Wfa, Wfb, bf)


def kernel(x, edge_index, batch, W1, b1, W2, b2, g1, be1, rm1, rv1,
           W3, b3, W4, b4, g2, be2, rm2, rv2, Wf, bf):
  src2d = edge_index[0].reshape(_NW * _NCHUNK, _CHUNK)
  dst2d = edge_index[1].reshape(_NW * _NCHUNK, _CHUNK)
  zeros32 = jnp.zeros((_NP, 32), jnp.float32)
  zeros16 = jnp.zeros((_NP, 16), jnp.float32)

  xp = _proj(x, W1)
  parts1 = _make_seg_sum(32)(xp, src2d, dst2d, zeros32)
  h = _mlp1(xp, parts1, b1.reshape(1, 32), W2, b2.reshape(1, 16),
            g1.reshape(1, 16), be1.reshape(1, 16), rm1.reshape(1, 16),
            rv1.reshape(1, 16))
  parts2 = _make_seg_sum(16)(h, src2d, dst2d, zeros16)
  h2 = _mlp2(h, parts2, W3, b3.reshape(1, 16), W4, b4.reshape(1, 16),
             g2.reshape(1, 16), be2.reshape(1, 16), rm2.reshape(1, 16),
             rv2.reshape(1, 16))
  bf32 = batch.astype(jnp.float32)
  bpacked = jnp.broadcast_to(bf32[:, None], (_N, 16)).reshape(_N // 8, 128)
  out = _pool(h2, h2.reshape(_N // 8, 128), bf32.reshape(1, _N), bpacked,
              Wf[:16], Wf[16:], bf.reshape(1, 2))
  return out


# speedup vs baseline: 1.1271x; 1.1271x over previous
import functools

import jax
import jax.numpy as jnp
from jax import lax
from jax.experimental import pallas as pl
from jax.experimental.pallas import tpu as pltpu
from jax.experimental.pallas import tpu_sc as plsc

_N = 10000
_E = 320000
_D = 128
_G = 64
_BN_EPS = 1e-5

_NC = 2
_NS = 16
_NW = _NC * _NS
_CHUNK = 125
_EPW = _E // _NW
_NCHUNK = _EPW // _CHUNK
_NP = 10240
_RPS = _NP // _NS


@functools.lru_cache(maxsize=None)
def _make_seg_sum(F):
  mesh = plsc.VectorSubcoreMesh(core_axis_name="c", subcore_axis_name="s",
                                num_cores=_NC, num_subcores=_NS)

  @functools.partial(
      pl.kernel,
      out_type=jax.ShapeDtypeStruct((_NC, _NP, F), jnp.float32),
      mesh=mesh,
      compiler_params=pltpu.CompilerParams(use_tc_tiling_on_sc=False),
      scratch_types=[
          pltpu.VMEM((_NCHUNK, _CHUNK), jnp.int32),
          pltpu.VMEM((_NCHUNK, _CHUNK), jnp.int32),
          pltpu.VMEM((4, _CHUNK, F), jnp.float32),
          pltpu.VMEM_SHARED((_NP, F), jnp.float32),
          pltpu.SemaphoreType.DMA,
          pltpu.SemaphoreType.DMA,
          pltpu.SemaphoreType.DMA,
          pltpu.SemaphoreType.DMA,
          pltpu.SemaphoreType.DMA,
          pltpu.SemaphoreType.DMA,
          pltpu.SemaphoreType.DMA,
          pltpu.SemaphoreType.DMA,
      ],
  )
  def seg_sum(feat, src2d, dst2d, zeros, out, src_v, dst_v, rows_v, acc,
              gs0, gs1, gs2, gs3, ss0, ss1, ss2, ss3):
    cid = lax.axis_index("c")
    sid = lax.axis_index("s")
    wid = sid * _NC + cid
    pltpu.sync_copy(zeros.at[pl.ds(sid * _RPS, _RPS)],
                    acc.at[pl.ds(sid * _RPS, _RPS)])
    pltpu.sync_copy(src2d.at[pl.ds(wid * _NCHUNK, _NCHUNK)], src_v)
    pltpu.sync_copy(dst2d.at[pl.ds(wid * _NCHUNK, _NCHUNK)], dst_v)
    plsc.subcore_barrier()

    nb = 4
    gsems = (gs0, gs1, gs2, gs3)
    ssems = (ss0, ss1, ss2, ss3)
    for b in range(nb - 1):
      pltpu.async_copy(feat.at[src_v.at[b]], rows_v.at[b], gsems[b])

    def body(j, carry):
      for b in range(nb):

        @pl.when(lax.rem(j, nb) == b)
        def _():
          pltpu.make_async_copy(feat.at[src_v.at[j]], rows_v.at[b],
                                gsems[b]).wait()
          pltpu.async_copy(rows_v.at[b], acc.at[dst_v.at[j]], ssems[b],
                           add=True)
          bn = (b + nb - 1) % nb
          jn = j + nb - 1

          @pl.when(jn < _NCHUNK)
          def _():

            @pl.when(j >= 1)
            def _():
              pltpu.make_async_copy(rows_v.at[bn], acc.at[dst_v.at[0]],
                                    ssems[bn]).wait()

            pltpu.async_copy(feat.at[src_v.at[jn]], rows_v.at[bn], gsems[bn])

      return carry

    lax.fori_loop(0, _NCHUNK, body, 0)
    for b in range(nb):
      pltpu.make_async_copy(rows_v.at[b], acc.at[dst_v.at[0]],
                            ssems[b]).wait()
    plsc.subcore_barrier()
    pltpu.sync_copy(acc.at[pl.ds(sid * _RPS, _RPS)],
                    out.at[cid, pl.ds(sid * _RPS, _RPS)])

  return seg_sum


def _proj_body(x_ref, w_ref, o_ref):
  o_ref[...] = jnp.dot(x_ref[...], w_ref[...],
                       preferred_element_type=jnp.float32, precision=jax.lax.Precision.HIGHEST)


def _proj(x, W1):
  return pl.pallas_call(
      _proj_body,
      grid=(10,),
      in_specs=[
          pl.BlockSpec((_N // 10, _D), lambda i: (i, 0)),
          pl.BlockSpec((_D, 32), lambda i: (0, 0)),
      ],
      out_specs=pl.BlockSpec((_N // 10, 32), lambda i: (i, 0)),
      out_shape=jax.ShapeDtypeStruct((_N, 32), jnp.float32),
  )(x, W1)


def _mlp1_body(xp_ref, parts_ref, b1_ref, w2_ref, b2_ref, g1_ref, be1_ref,
               rm1_ref, rv1_ref, o_ref):
  t = xp_ref[...] + parts_ref[0] + parts_ref[1] + b1_ref[...]
  t = jnp.maximum(t, 0.0)
  t = jnp.dot(t, w2_ref[...], preferred_element_type=jnp.float32, precision=jax.lax.Precision.HIGHEST) + b2_ref[...]
  t = jnp.maximum(t, 0.0)
  scale = g1_ref[...] * lax.rsqrt(rv1_ref[...] + _BN_EPS)
  t = (t - rm1_ref[...]) * scale + be1_ref[...]
  o_ref[...] = jnp.maximum(t, 0.0)


def _mlp1(xp, parts, b1, W2, b2, g1, be1, rm1, rv1):
  blk = _N // 10
  return pl.pallas_call(
      _mlp1_body,
      grid=(10,),
      in_specs=[
          pl.BlockSpec((blk, 32), lambda i: (i, 0)),
          pl.BlockSpec((_NC, blk, 32), lambda i: (0, i, 0)),
          pl.BlockSpec((1, 32), lambda i: (0, 0)),
          pl.BlockSpec((32, 16), lambda i: (0, 0)),
          pl.BlockSpec((1, 16), lambda i: (0, 0)),
          pl.BlockSpec((1, 16), lambda i: (0, 0)),
          pl.BlockSpec((1, 16), lambda i: (0, 0)),
          pl.BlockSpec((1, 16), lambda i: (0, 0)),
          pl.BlockSpec((1, 16), lambda i: (0, 0)),
      ],
      out_specs=pl.BlockSpec((blk, 16), lambda i: (i, 0)),
      out_shape=jax.ShapeDtypeStruct((_N, 16), jnp.float32),
  )(xp, parts, b1, W2, b2, g1, be1, rm1, rv1)


def _mlp2_body(h_ref, parts_ref, w3_ref, b3_ref, w4_ref, b4_ref, g2_ref,
               be2_ref, rm2_ref, rv2_ref, o_ref):
  t = h_ref[...] + parts_ref[0] + parts_ref[1]
  t = jnp.dot(t, w3_ref[...], preferred_element_type=jnp.float32, precision=jax.lax.Precision.HIGHEST) + b3_ref[...]
  t = jnp.maximum(t, 0.0)
  t = jnp.dot(t, w4_ref[...], preferred_element_type=jnp.float32, precision=jax.lax.Precision.HIGHEST) + b4_ref[...]
  t = jnp.maximum(t, 0.0)
  scale = g2_ref[...] * lax.rsqrt(rv2_ref[...] + _BN_EPS)
  t = (t - rm2_ref[...]) * scale + be2_ref[...]
  o_ref[...] = jnp.maximum(t, 0.0)


def _mlp2(h, parts, W3, b3, W4, b4, g2, be2, rm2, rv2):
  blk = _N // 10
  return pl.pallas_call(
      _mlp2_body,
      grid=(10,),
      in_specs=[
          pl.BlockSpec((blk, 16), lambda i: (i, 0)),
          pl.BlockSpec((_NC, blk, 16), lambda i: (0, i, 0)),
          pl.BlockSpec((16, 16), lambda i: (0, 0)),
          pl.BlockSpec((1, 16), lambda i: (0, 0)),
          pl.BlockSpec((16, 16), lambda i: (0, 0)),
          pl.BlockSpec((1, 16), lambda i: (0, 0)),
          pl.BlockSpec((1, 16), lambda i: (0, 0)),
          pl.BlockSpec((1, 16), lambda i: (0, 0)),
          pl.BlockSpec((1, 16), lambda i: (0, 0)),
          pl.BlockSpec((1, 16), lambda i: (0, 0)),
      ],
      out_specs=pl.BlockSpec((blk, 16), lambda i: (i, 0)),
      out_shape=jax.ShapeDtypeStruct((_N, 16), jnp.float32),
  )(h, parts, W3, b3, W4, b4, g2, be2, rm2, rv2)


def _pool_body(h2_ref, h2p_ref, bt_ref, bp_ref, wfa_ref, wfb_ref, bf_ref,
               o_ref):
  hp = jax.lax.Precision.HIGHEST
  h2 = h2_ref[...]
  h2p = h2p_ref[...]
  bt = bt_ref[...]
  bp = bp_ref[...]
  rows = lax.broadcasted_iota(jnp.int32, (_G, 1), 0)

  oht = (lax.broadcasted_iota(jnp.int32, (_G, _N), 0).astype(jnp.float32)
         == bt).astype(jnp.float32)
  xsum = jnp.dot(oht, h2, preferred_element_type=jnp.float32, precision=hp)
  cnt = jnp.dot(oht, jnp.ones((_N, 1), jnp.float32),
                preferred_element_type=jnp.float32, precision=hp)
  xmean = xsum / jnp.maximum(cnt, 1.0)

  def body(g, xmax):
    t = jnp.where(bp == g.astype(jnp.float32), h2p, -jnp.inf)
    r = jnp.max(t, axis=0, keepdims=True)
    r = jnp.maximum(r[:, :64], r[:, 64:])
    r = jnp.maximum(r[:, :32], r[:, 32:])
    r = jnp.maximum(r[:, :16], r[:, 16:])
    return jnp.where(rows == g, r, xmax)

  xmax = lax.fori_loop(0, _G, body,
                       jnp.full((_G, 16), -jnp.inf, jnp.float32))
  o_ref[...] = (jnp.dot(xmax, wfa_ref[...],
                        preferred_element_type=jnp.float32, precision=hp)
                + jnp.dot(xmean, wfb_ref[...],
                          preferred_element_type=jnp.float32, precision=hp)
                + bf_ref[...])


def _pool(h2, h2p, batcht, batchp, Wfa, Wfb, bf):
  return pl.pallas_call(
      _pool_body,
      out_shape=jax.ShapeDtypeStruct((_G, 2), jnp.float32),
  )(h2, h2p, batcht, batchp, Wfa, Wfb, bf)


def kernel(x, edge_index, batch, W1, b1, W2, b2, g1, be1, rm1, rv1,
           W3, b3, W4, b4, g2, be2, rm2, rv2, Wf, bf):
  src2d = edge_index[0].reshape(_NW * _NCHUNK, _CHUNK)
  dst2d = edge_index[1].reshape(_NW * _NCHUNK, _CHUNK)
  zeros32 = jnp.zeros((_NP, 32), jnp.float32)
  zeros16 = jnp.zeros((_NP, 16), jnp.float32)

  xp = _proj(x, W1)
  parts1 = _make_seg_sum(32)(xp, src2d, dst2d, zeros32)
  h = _mlp1(xp, parts1, b1.reshape(1, 32), W2, b2.reshape(1, 16),
            g1.reshape(1, 16), be1.reshape(1, 16), rm1.reshape(1, 16),
            rv1.reshape(1, 16))
  parts2 = _make_seg_sum(16)(h, src2d, dst2d, zeros16)
  h2 = _mlp2(h, parts2, W3, b3.reshape(1, 16), W4, b4.reshape(1, 16),
             g2.reshape(1, 16), be2.reshape(1, 16), rm2.reshape(1, 16),
             rv2.reshape(1, 16))
  bf32 = batch.astype(jnp.float32)
  bpacked = jnp.broadcast_to(bf32[:, None], (_N, 16)).reshape(_N // 8, 128)
  out = _pool(h2, h2.reshape(_N // 8, 128), bf32.reshape(1, _N), bpacked,
              Wf[:16], Wf[16:], bf.reshape(1, 2))
  return out


# speedup vs baseline: 1.1828x; 1.0494x over previous
import functools

import jax
import jax.numpy as jnp
from jax import lax
from jax.experimental import pallas as pl
from jax.experimental.pallas import tpu as pltpu
from jax.experimental.pallas import tpu_sc as plsc

_N = 10000
_E = 320000
_D = 128
_G = 64
_BN_EPS = 1e-5

_NC = 2
_NS = 16
_NW = _NC * _NS
_CHUNK = 125
_EPW = _E // _NW
_NCHUNK = _EPW // _CHUNK
_NP = 10240
_RPS = _NP // _NS


@functools.lru_cache(maxsize=None)
def _make_seg_sum(F):
  mesh = plsc.VectorSubcoreMesh(core_axis_name="c", subcore_axis_name="s",
                                num_cores=_NC, num_subcores=_NS)

  @functools.partial(
      pl.kernel,
      out_type=jax.ShapeDtypeStruct((_NC, _NP, F), jnp.float32),
      mesh=mesh,
      compiler_params=pltpu.CompilerParams(use_tc_tiling_on_sc=False),
      scratch_types=[
          pltpu.VMEM((_NCHUNK, _CHUNK), jnp.int32),
          pltpu.VMEM((_NCHUNK, _CHUNK), jnp.int32),
          pltpu.VMEM((6, _CHUNK, F), jnp.float32),
          pltpu.VMEM_SHARED((_NP, F), jnp.float32),
          pltpu.SemaphoreType.DMA,
          pltpu.SemaphoreType.DMA,
          pltpu.SemaphoreType.DMA,
          pltpu.SemaphoreType.DMA,
          pltpu.SemaphoreType.DMA,
          pltpu.SemaphoreType.DMA,
          pltpu.SemaphoreType.DMA,
          pltpu.SemaphoreType.DMA,
          pltpu.SemaphoreType.DMA,
          pltpu.SemaphoreType.DMA,
          pltpu.SemaphoreType.DMA,
          pltpu.SemaphoreType.DMA,
      ],
  )
  def seg_sum(feat, src2d, dst2d, zeros, out, src_v, dst_v, rows_v, acc,
              gs0, gs1, gs2, gs3, gs4, gs5, ss0, ss1, ss2, ss3, ss4, ss5):
    cid = lax.axis_index("c")
    sid = lax.axis_index("s")
    wid = sid * _NC + cid
    pltpu.sync_copy(zeros.at[pl.ds(sid * _RPS, _RPS)],
                    acc.at[pl.ds(sid * _RPS, _RPS)])
    pltpu.sync_copy(src2d.at[pl.ds(wid * _NCHUNK, _NCHUNK)], src_v)
    pltpu.sync_copy(dst2d.at[pl.ds(wid * _NCHUNK, _NCHUNK)], dst_v)
    plsc.subcore_barrier()

    nb = 6
    gsems = (gs0, gs1, gs2, gs3, gs4, gs5)
    ssems = (ss0, ss1, ss2, ss3, ss4, ss5)
    for b in range(nb - 1):
      pltpu.async_copy(feat.at[src_v.at[b]], rows_v.at[b], gsems[b])

    def body(j, carry):
      for b in range(nb):

        @pl.when(lax.rem(j, nb) == b)
        def _():
          pltpu.make_async_copy(feat.at[src_v.at[j]], rows_v.at[b],
                                gsems[b]).wait()
          pltpu.async_copy(rows_v.at[b], acc.at[dst_v.at[j]], ssems[b],
                           add=True)
          bn = (b + nb - 1) % nb
          jn = j + nb - 1

          @pl.when(jn < _NCHUNK)
          def _():

            @pl.when(j >= 1)
            def _():
              pltpu.make_async_copy(rows_v.at[bn], acc.at[dst_v.at[0]],
                                    ssems[bn]).wait()

            pltpu.async_copy(feat.at[src_v.at[jn]], rows_v.at[bn], gsems[bn])

      return carry

    lax.fori_loop(0, _NCHUNK, body, 0)
    for b in range(nb):
      pltpu.make_async_copy(rows_v.at[b], acc.at[dst_v.at[0]],
                            ssems[b]).wait()
    plsc.subcore_barrier()
    pltpu.sync_copy(acc.at[pl.ds(sid * _RPS, _RPS)],
                    out.at[cid, pl.ds(sid * _RPS, _RPS)])

  return seg_sum


def _proj_body(x_ref, w_ref, o_ref):
  o_ref[...] = jnp.dot(x_ref[...], w_ref[...],
                       preferred_element_type=jnp.float32, precision=jax.lax.Precision.HIGHEST)


def _proj(x, W1):
  return pl.pallas_call(
      _proj_body,
      grid=(10,),
      in_specs=[
          pl.BlockSpec((_N // 10, _D), lambda i: (i, 0)),
          pl.BlockSpec((_D, 32), lambda i: (0, 0)),
      ],
      out_specs=pl.BlockSpec((_N // 10, 32), lambda i: (i, 0)),
      out_shape=jax.ShapeDtypeStruct((_N, 32), jnp.float32),
  )(x, W1)


def _mlp1_body(xp_ref, parts_ref, b1_ref, w2_ref, b2_ref, g1_ref, be1_ref,
               rm1_ref, rv1_ref, o_ref):
  t = xp_ref[...] + parts_ref[0] + parts_ref[1] + b1_ref[...]
  t = jnp.maximum(t, 0.0)
  t = jnp.dot(t, w2_ref[...], preferred_element_type=jnp.float32, precision=jax.lax.Precision.HIGHEST) + b2_ref[...]
  t = jnp.maximum(t, 0.0)
  scale = g1_ref[...] * lax.rsqrt(rv1_ref[...] + _BN_EPS)
  t = (t - rm1_ref[...]) * scale + be1_ref[...]
  o_ref[...] = jnp.maximum(t, 0.0)


def _mlp1(xp, parts, b1, W2, b2, g1, be1, rm1, rv1):
  blk = _N // 10
  return pl.pallas_call(
      _mlp1_body,
      grid=(10,),
      in_specs=[
          pl.BlockSpec((blk, 32), lambda i: (i, 0)),
          pl.BlockSpec((_NC, blk, 32), lambda i: (0, i, 0)),
          pl.BlockSpec((1, 32), lambda i: (0, 0)),
          pl.BlockSpec((32, 16), lambda i: (0, 0)),
          pl.BlockSpec((1, 16), lambda i: (0, 0)),
          pl.BlockSpec((1, 16), lambda i: (0, 0)),
          pl.BlockSpec((1, 16), lambda i: (0, 0)),
          pl.BlockSpec((1, 16), lambda i: (0, 0)),
          pl.BlockSpec((1, 16), lambda i: (0, 0)),
      ],
      out_specs=pl.BlockSpec((blk, 16), lambda i: (i, 0)),
      out_shape=jax.ShapeDtypeStruct((_N, 16), jnp.float32),
  )(xp, parts, b1, W2, b2, g1, be1, rm1, rv1)


def _mlp2_body(h_ref, parts_ref, w3_ref, b3_ref, w4_ref, b4_ref, g2_ref,
               be2_ref, rm2_ref, rv2_ref, o_ref):
  t = h_ref[...] + parts_ref[0] + parts_ref[1]
  t = jnp.dot(t, w3_ref[...], preferred_element_type=jnp.float32, precision=jax.lax.Precision.HIGHEST) + b3_ref[...]
  t = jnp.maximum(t, 0.0)
  t = jnp.dot(t, w4_ref[...], preferred_element_type=jnp.float32, precision=jax.lax.Precision.HIGHEST) + b4_ref[...]
  t = jnp.maximum(t, 0.0)
  scale = g2_ref[...] * lax.rsqrt(rv2_ref[...] + _BN_EPS)
  t = (t - rm2_ref[...]) * scale + be2_ref[...]
  o_ref[...] = jnp.maximum(t, 0.0)


def _mlp2(h, parts, W3, b3, W4, b4, g2, be2, rm2, rv2):
  blk = _N // 10
  return pl.pallas_call(
      _mlp2_body,
      grid=(10,),
      in_specs=[
          pl.BlockSpec((blk, 16), lambda i: (i, 0)),
          pl.BlockSpec((_NC, blk, 16), lambda i: (0, i, 0)),
          pl.BlockSpec((16, 16), lambda i: (0, 0)),
          pl.BlockSpec((1, 16), lambda i: (0, 0)),
          pl.BlockSpec((16, 16), lambda i: (0, 0)),
          pl.BlockSpec((1, 16), lambda i: (0, 0)),
          pl.BlockSpec((1, 16), lambda i: (0, 0)),
          pl.BlockSpec((1, 16), lambda i: (0, 0)),
          pl.BlockSpec((1, 16), lambda i: (0, 0)),
          pl.BlockSpec((1, 16), lambda i: (0, 0)),
      ],
      out_specs=pl.BlockSpec((blk, 16), lambda i: (i, 0)),
      out_shape=jax.ShapeDtypeStruct((_N, 16), jnp.float32),
  )(h, parts, W3, b3, W4, b4, g2, be2, rm2, rv2)


def _pool_body(h2_ref, h2p_ref, bt_ref, bp_ref, wfa_ref, wfb_ref, bf_ref,
               o_ref):
  hp = jax.lax.Precision.HIGHEST
  h2 = h2_ref[...]
  h2p = h2p_ref[...]
  bt = bt_ref[...]
  bp = bp_ref[...]
  rows = lax.broadcasted_iota(jnp.int32, (_G, 1), 0)

  oht = (lax.broadcasted_iota(jnp.int32, (_G, _N), 0).astype(jnp.float32)
         == bt).astype(jnp.float32)
  xsum = jnp.dot(oht, h2, preferred_element_type=jnp.float32, precision=hp)
  cnt = jnp.dot(oht, jnp.ones((_N, 1), jnp.float32),
                preferred_element_type=jnp.float32, precision=hp)
  xmean = xsum / jnp.maximum(cnt, 1.0)

  def body(g, xmax):
    t = jnp.where(bp == g.astype(jnp.float32), h2p, -jnp.inf)
    r = jnp.max(t, axis=0, keepdims=True)
    r = jnp.maximum(r[:, :64], r[:, 64:])
    r = jnp.maximum(r[:, :32], r[:, 32:])
    r = jnp.maximum(r[:, :16], r[:, 16:])
    return jnp.where(rows == g, r, xmax)

  xmax = lax.fori_loop(0, _G, body,
                       jnp.full((_G, 16), -jnp.inf, jnp.float32))
  o_ref[...] = (jnp.dot(xmax, wfa_ref[...],
                        preferred_element_type=jnp.float32, precision=hp)
                + jnp.dot(xmean, wfb_ref[...],
                          preferred_element_type=jnp.float32, precision=hp)
                + bf_ref[...])


def _pool(h2, h2p, batcht, batchp, Wfa, Wfb, bf):
  return pl.pallas_call(
      _pool_body,
      out_shape=jax.ShapeDtypeStruct((_G, 2), jnp.float32),
  )(h2, h2p, batcht, batchp, Wfa, Wfb, bf)


def kernel(x, edge_index, batch, W1, b1, W2, b2, g1, be1, rm1, rv1,
           W3, b3, W4, b4, g2, be2, rm2, rv2, Wf, bf):
  src2d = edge_index[0].reshape(_NW * _NCHUNK, _CHUNK)
  dst2d = edge_index[1].reshape(_NW * _NCHUNK, _CHUNK)
  zeros32 = jnp.zeros((_NP, 32), jnp.float32)
  zeros16 = jnp.zeros((_NP, 16), jnp.float32)

  xp = _proj(x, W1)
  parts1 = _make_seg_sum(32)(xp, src2d, dst2d, zeros32)
  h = _mlp1(xp, parts1, b1.reshape(1, 32), W2, b2.reshape(1, 16),
            g1.reshape(1, 16), be1.reshape(1, 16), rm1.reshape(1, 16),
            rv1.reshape(1, 16))
  parts2 = _make_seg_sum(16)(h, src2d, dst2d, zeros16)
  h2 = _mlp2(h, parts2, W3, b3.reshape(1, 16), W4, b4.reshape(1, 16),
             g2.reshape(1, 16), be2.reshape(1, 16), rm2.reshape(1, 16),
             rv2.reshape(1, 16))
  bf32 = batch.astype(jnp.float32)
  bpacked = jnp.broadcast_to(bf32[:, None], (_N, 16)).reshape(_N // 8, 128)
  out = _pool(h2, h2.reshape(_N // 8, 128), bf32.reshape(1, _N), bpacked,
              Wf[:16], Wf[16:], bf.reshape(1, 2))
  return out
